# baseline (speedup 1.0000x reference)
import jax
import jax.numpy as jnp
from jax.experimental import pallas as pl
from jax.experimental.pallas import tpu as pltpu

B = 64
DIM_CODES = 256
BOOK = 1024
EMB = 64
ZDIM = 256
NPSEUDO = 16
HID = 128
W_DIM = DIM_CODES * EMB
P = DIM_CODES // 2

D_BLK = 16
N_BLK = DIM_CODES // D_BLK


def _fused_kernel(wq_ref, w0_ref, w1_ref, w2_ref, cbias_ref, ew_ref, eb_ref,
                  d1w_ref, d1b_ref, d2w_ref, d2b_ref, cbk_ref,
                  wrec_ref, dist_ref, idx_ref, h_scr):
    i = pl.program_id(0)
    dn = (((1,), (1,)), ((), ()))

    @pl.when(i == 0)
    def _prologue():
        wq = wq_ref[...]
        even = wq[:, :EMB]
        evs = jnp.concatenate([even[1:], jnp.zeros((1, EMB), jnp.float32)], axis=0)
        row = jax.lax.broadcasted_iota(jnp.int32, (B * P, 1), 0)
        evs = jnp.where(row % P != P - 1, evs, 0.0)
        xx = jnp.concatenate([wq, evs], axis=1)
        ww = jnp.concatenate([w0_ref[...], w1_ref[...], w2_ref[...]], axis=1)
        y = jax.lax.dot_general(xx, ww, dn, preferred_element_type=jnp.float32)
        y = jnp.maximum(y + cbias_ref[...], 0.0)
        y2 = y.reshape(B, P, HID).transpose(0, 2, 1).reshape(B, P * HID)
        mu = jax.lax.dot_general(y2, ew_ref[...], dn,
                                 preferred_element_type=jnp.float32) + eb_ref[...]
        h = jax.lax.dot_general(mu, d1w_ref[...], dn,
                                preferred_element_type=jnp.float32) + d1b_ref[...]
        h_scr[...] = jnp.maximum(h, 0.0)

    @pl.when(i > 0)
    def _dist_block():
        h = h_scr[...]
        wrec = jax.lax.dot_general(h, d2w_ref[...], dn,
                                   preferred_element_type=jnp.float32) + d2b_ref[...]
        wrec_ref[...] = wrec
        k_iota = jax.lax.broadcasted_iota(jnp.int32, (B, BOOK), 1)
        cols = []
        for j in range(D_BLK):
            xr = wrec[:, j * EMB:(j + 1) * EMB]
            cb = cbk_ref[j]
            xcm2 = jax.lax.dot_general(xr * (-2.0), cb, dn,
                                       preferred_element_type=jnp.float32)
            x2 = jnp.sum(xr * xr, axis=1, keepdims=True)
            c2 = jnp.sum(cb * cb, axis=1)[None, :]
            dist = (x2 + c2) + xcm2
            dist_ref[:, j * BOOK:(j + 1) * BOOK] = dist
            mn = jnp.min(dist, axis=1, keepdims=True)
            idxj = jnp.min(jnp.where(dist == mn, k_iota, BOOK), axis=1,
                           keepdims=True)
            cols.append(idxj)
        idx_ref[0] = jnp.concatenate(cols, axis=1)


def _shift(i):
    return jax.lax.max(i - 1, 0)


@jax.jit
def kernel(w_q, pseudo_inputs, conv_w, conv_b, enc_w, enc_b,
           dec1_w, dec1_b, dec2_w, dec2_b, codebook):
    f32 = jnp.float32
    wq128 = w_q.reshape(B * P, 2 * EMB)
    w0 = conv_w[:, :, 0]
    w1 = conv_w[:, :, 1]
    w2 = conv_w[:, :, 2]

    wrec, dist, idx = pl.pallas_call(
        _fused_kernel,
        grid=(N_BLK + 1,),
        in_specs=[
            pl.BlockSpec((B * P, 2 * EMB), lambda i: (0, 0)),
            pl.BlockSpec((HID, EMB), lambda i: (0, 0)),
            pl.BlockSpec((HID, EMB), lambda i: (0, 0)),
            pl.BlockSpec((HID, EMB), lambda i: (0, 0)),
            pl.BlockSpec((1, HID), lambda i: (0, 0)),
            pl.BlockSpec((ZDIM, P * HID), lambda i: (0, 0)),
            pl.BlockSpec((1, ZDIM), lambda i: (0, 0)),
            pl.BlockSpec((1024, ZDIM), lambda i: (0, 0)),
            pl.BlockSpec((1, 1024), lambda i: (0, 0)),
            pl.BlockSpec((D_BLK * EMB, 1024), lambda i: (_shift(i), 0)),
            pl.BlockSpec((1, D_BLK * EMB), lambda i: (0, _shift(i))),
            pl.BlockSpec((D_BLK, BOOK, EMB), lambda i: (_shift(i), 0, 0)),
        ],
        out_specs=[
            pl.BlockSpec((B, D_BLK * EMB), lambda i: (0, _shift(i))),
            pl.BlockSpec((B, D_BLK * BOOK), lambda i: (0, _shift(i))),
            pl.BlockSpec((1, B, D_BLK), lambda i: (_shift(i), 0, 0)),
        ],
        out_shape=[
            jax.ShapeDtypeStruct((B, W_DIM), f32),
            jax.ShapeDtypeStruct((B, DIM_CODES * BOOK), f32),
            jax.ShapeDtypeStruct((N_BLK, B, D_BLK), jnp.int32),
        ],
        scratch_shapes=[pltpu.VMEM((B, 1024), f32)],
    )(wq128, w0, w1, w2, conv_b.reshape(1, HID), enc_w,
      enc_b.reshape(1, 2 * ZDIM), dec1_w, dec1_b.reshape(1, 1024),
      dec2_w, dec2_b.reshape(1, W_DIM), codebook)

    idx = idx.transpose(1, 0, 2).reshape(B, DIM_CODES)
    return wrec, dist.reshape(B, DIM_CODES, BOOK), idx[:, :, None]


# speedup vs baseline: 1.1940x; 1.1940x over previous
import jax
import jax.numpy as jnp
from jax.experimental import pallas as pl
from jax.experimental.pallas import tpu as pltpu

B = 64
DIM_CODES = 256
BOOK = 1024
EMB = 64
ZDIM = 256
NPSEUDO = 16
HID = 128
W_DIM = DIM_CODES * EMB
P = DIM_CODES // 2

D_BLK = 16
N_BLK = DIM_CODES // D_BLK


def _fused_kernel(wq_ref, w0_ref, w1_ref, w2_ref, cbias_ref, ew_ref, eb_ref,
                  d1w_ref, d1b_ref, d2w_ref, d2b_ref, cbk_ref,
                  wrec_ref, dist_ref, idx_ref, h_scr):
    i = pl.program_id(0)
    dn = (((1,), (1,)), ((), ()))

    @pl.when(i == 0)
    def _prologue():
        wq = wq_ref[...]
        even = wq[:, :EMB]
        evs = jnp.concatenate([even[1:], jnp.zeros((1, EMB), jnp.float32)], axis=0)
        row = jax.lax.broadcasted_iota(jnp.int32, (B * P, 1), 0)
        evs = jnp.where(row % P != P - 1, evs, 0.0)
        xx = jnp.concatenate([wq, evs], axis=1)
        ww = jnp.concatenate([w0_ref[...], w1_ref[...], w2_ref[...]], axis=1)
        y = jax.lax.dot_general(xx, ww, dn, preferred_element_type=jnp.float32)
        y = jnp.maximum(y + cbias_ref[...], 0.0)
        y2 = y.reshape(B, P, HID).transpose(0, 2, 1).reshape(B, P * HID)
        mu = jax.lax.dot_general(y2, ew_ref[...], dn,
                                 preferred_element_type=jnp.float32) + eb_ref[...]
        h = jax.lax.dot_general(mu, d1w_ref[...], dn,
                                preferred_element_type=jnp.float32) + d1b_ref[...]
        h_scr[...] = jnp.maximum(h, 0.0)

    @pl.when(i > 0)
    def _dist_block():
        h = h_scr[...]
        wrec = jax.lax.dot_general(h, d2w_ref[...], dn,
                                   preferred_element_type=jnp.float32) + d2b_ref[...]
        wrec_ref[...] = wrec
        k_iota = jax.lax.broadcasted_iota(jnp.int32, (B, BOOK), 1)
        cols = []
        for j in range(D_BLK):
            xr = wrec[:, j * EMB:(j + 1) * EMB]
            cb = cbk_ref[j]
            xcm2 = jax.lax.dot_general(xr * (-2.0), cb, dn,
                                       preferred_element_type=jnp.float32)
            x2 = jnp.sum(xr * xr, axis=1, keepdims=True)
            c2 = jnp.sum(cb * cb, axis=1)[None, :]
            dist = (x2 + c2) + xcm2
            dist_ref[:, j, :] = dist
            mn = jnp.min(dist, axis=1, keepdims=True)
            idxj = jnp.min(jnp.where(dist == mn, k_iota, BOOK), axis=1,
                           keepdims=True)
            cols.append(idxj)
        idx_ref[_shift(i)] = jnp.concatenate(cols, axis=1)


def _shift(i):
    return jax.lax.max(i - 1, 0)


@jax.jit
def kernel(w_q, pseudo_inputs, conv_w, conv_b, enc_w, enc_b,
           dec1_w, dec1_b, dec2_w, dec2_b, codebook):
    f32 = jnp.float32
    wq128 = w_q.reshape(B * P, 2 * EMB)
    w0 = conv_w[:, :, 0]
    w1 = conv_w[:, :, 1]
    w2 = conv_w[:, :, 2]

    wrec, dist, idx = pl.pallas_call(
        _fused_kernel,
        grid=(N_BLK + 1,),
        in_specs=[
            pl.BlockSpec((B * P, 2 * EMB), lambda i: (0, 0)),
            pl.BlockSpec((HID, EMB), lambda i: (0, 0)),
            pl.BlockSpec((HID, EMB), lambda i: (0, 0)),
            pl.BlockSpec((HID, EMB), lambda i: (0, 0)),
            pl.BlockSpec((1, HID), lambda i: (0, 0)),
            pl.BlockSpec((ZDIM, P * HID), lambda i: (0, 0)),
            pl.BlockSpec((1, ZDIM), lambda i: (0, 0)),
            pl.BlockSpec((1024, ZDIM), lambda i: (0, 0)),
            pl.BlockSpec((1, 1024), lambda i: (0, 0)),
            pl.BlockSpec((D_BLK * EMB, 1024), lambda i: (_shift(i), 0)),
            pl.BlockSpec((1, D_BLK * EMB), lambda i: (0, _shift(i))),
            pl.BlockSpec((D_BLK, BOOK, EMB), lambda i: (_shift(i), 0, 0)),
        ],
        out_specs=[
            pl.BlockSpec((B, D_BLK * EMB), lambda i: (0, _shift(i))),
            pl.BlockSpec((B, D_BLK, BOOK), lambda i: (0, _shift(i), 0)),
            pl.BlockSpec((N_BLK, B, D_BLK), lambda i: (0, 0, 0)),
        ],
        out_shape=[
            jax.ShapeDtypeStruct((B, W_DIM), f32),
            jax.ShapeDtypeStruct((B, DIM_CODES, BOOK), f32),
            jax.ShapeDtypeStruct((N_BLK, B, D_BLK), jnp.int32),
        ],
        scratch_shapes=[pltpu.VMEM((B, 1024), f32)],
    )(wq128, w0, w1, w2, conv_b.reshape(1, HID), enc_w,
      enc_b.reshape(1, 2 * ZDIM), dec1_w, dec1_b.reshape(1, 1024),
      dec2_w, dec2_b.reshape(1, W_DIM), codebook)

    idx = idx.transpose(1, 0, 2).reshape(B, DIM_CODES)
    return wrec, dist, idx[:, :, None]


# speedup vs baseline: 1.2396x; 1.0382x over previous
import jax
import jax.numpy as jnp
from jax.experimental import pallas as pl
from jax.experimental.pallas import tpu as pltpu

B = 64
DIM_CODES = 256
BOOK = 1024
EMB = 64
ZDIM = 256
NPSEUDO = 16
HID = 128
W_DIM = DIM_CODES * EMB
P = DIM_CODES // 2

D_BLK = 8
N_BLK = DIM_CODES // D_BLK


def _fused_kernel(wq_ref, w0_ref, w1_ref, w2_ref, cbias_ref, ew_ref, eb_ref,
                  d1w_ref, d1b_ref, d2w_ref, d2b_ref, cbk_ref,
                  wrec_ref, dist_ref, idx_ref, h_scr):
    i = pl.program_id(0)
    dn = (((1,), (1,)), ((), ()))

    @pl.when(i == 0)
    def _prologue():
        wq = wq_ref[...]
        even = wq[:, :EMB]
        evs = jnp.concatenate([even[1:], jnp.zeros((1, EMB), jnp.float32)], axis=0)
        row = jax.lax.broadcasted_iota(jnp.int32, (B * P, 1), 0)
        evs = jnp.where(row % P != P - 1, evs, 0.0)
        xx = jnp.concatenate([wq, evs], axis=1)
        ww = jnp.concatenate([w0_ref[...], w1_ref[...], w2_ref[...]], axis=1)
        y = jax.lax.dot_general(xx, ww, dn, preferred_element_type=jnp.float32)
        y = jnp.maximum(y + cbias_ref[...], 0.0)
        y2 = y.reshape(B, P, HID).transpose(0, 2, 1).reshape(B, P * HID)
        mu = jax.lax.dot_general(y2, ew_ref[...], dn,
                                 preferred_element_type=jnp.float32) + eb_ref[...]
        h = jax.lax.dot_general(mu, d1w_ref[...], dn,
                                preferred_element_type=jnp.float32) + d1b_ref[...]
        h_scr[...] = jnp.maximum(h, 0.0)

    @pl.when(i > 0)
    def _dist_block():
        h = h_scr[...]
        wrec = jax.lax.dot_general(h, d2w_ref[...], dn,
                                   preferred_element_type=jnp.float32) + d2b_ref[...]
        wrec_ref[...] = wrec
        k_iota = jax.lax.broadcasted_iota(jnp.int32, (B, BOOK), 1)
        cols = []
        for j in range(D_BLK):
            xr = wrec[:, j * EMB:(j + 1) * EMB]
            cb = cbk_ref[j]
            xcm2 = jax.lax.dot_general(xr * (-2.0), cb, dn,
                                       preferred_element_type=jnp.float32)
            x2 = jnp.sum(xr * xr, axis=1, keepdims=True)
            c2 = jnp.sum(cb * cb, axis=1)[None, :]
            dist = (x2 + c2) + xcm2
            dist_ref[:, j, :] = dist
            mn = jnp.min(dist, axis=1, keepdims=True)
            idxj = jnp.min(jnp.where(dist == mn, k_iota, BOOK), axis=1,
                           keepdims=True)
            cols.append(idxj)
        idx_ref[_shift(i)] = jnp.concatenate(cols, axis=1)


def _shift(i):
    return jax.lax.max(i - 1, 0)


@jax.jit
def kernel(w_q, pseudo_inputs, conv_w, conv_b, enc_w, enc_b,
           dec1_w, dec1_b, dec2_w, dec2_b, codebook):
    f32 = jnp.float32
    wq128 = w_q.reshape(B * P, 2 * EMB)
    w0 = conv_w[:, :, 0]
    w1 = conv_w[:, :, 1]
    w2 = conv_w[:, :, 2]

    wrec, dist, idx = pl.pallas_call(
        _fused_kernel,
        grid=(N_BLK + 1,),
        in_specs=[
            pl.BlockSpec((B * P, 2 * EMB), lambda i: (0, 0)),
            pl.BlockSpec((HID, EMB), lambda i: (0, 0)),
            pl.BlockSpec((HID, EMB), lambda i: (0, 0)),
            pl.BlockSpec((HID, EMB), lambda i: (0, 0)),
            pl.BlockSpec((1, HID), lambda i: (0, 0)),
            pl.BlockSpec((ZDIM, P * HID), lambda i: (0, 0)),
            pl.BlockSpec((1, ZDIM), lambda i: (0, 0)),
            pl.BlockSpec((1024, ZDIM), lambda i: (0, 0)),
            pl.BlockSpec((1, 1024), lambda i: (0, 0)),
            pl.BlockSpec((D_BLK * EMB, 1024), lambda i: (_shift(i), 0)),
            pl.BlockSpec((1, D_BLK * EMB), lambda i: (0, _shift(i))),
            pl.BlockSpec((D_BLK, BOOK, EMB), lambda i: (_shift(i), 0, 0)),
        ],
        out_specs=[
            pl.BlockSpec((B, D_BLK * EMB), lambda i: (0, _shift(i))),
            pl.BlockSpec((B, D_BLK, BOOK), lambda i: (0, _shift(i), 0)),
            pl.BlockSpec((N_BLK, B, D_BLK), lambda i: (0, 0, 0)),
        ],
        out_shape=[
            jax.ShapeDtypeStruct((B, W_DIM), f32),
            jax.ShapeDtypeStruct((B, DIM_CODES, BOOK), f32),
            jax.ShapeDtypeStruct((N_BLK, B, D_BLK), jnp.int32),
        ],
        scratch_shapes=[pltpu.VMEM((B, 1024), f32)],
    )(wq128, w0, w1, w2, conv_b.reshape(1, HID), enc_w,
      enc_b.reshape(1, 2 * ZDIM), dec1_w, dec1_b.reshape(1, 1024),
      dec2_w, dec2_b.reshape(1, W_DIM), codebook)

    idx = idx.transpose(1, 0, 2).reshape(B, DIM_CODES)
    return wrec, dist, idx[:, :, None]
